# baseline (speedup 1.0000x reference)
import functools

import jax
import jax.numpy as jnp
from jax import lax
from jax.experimental import pallas as pl
from jax.experimental.pallas import tpu as pltpu
from jax.experimental.pallas import tpu_sc as plsc

_BETA_SQUARED = 1.0

_NC = 2
_NS = 16
_NW = _NC * _NS
_L = 16

_C = 512
_ROWS = 8192

_SC_ROWS = 3072
_TCA_ROWS = 3584
_TCB_ROWS = _ROWS - _SC_ROWS - _TCA_ROWS

_PW = _SC_ROWS // _NW
_NPH = 2
_PH_ROWS = _PW // _NPH

_GRP = 8
_NG = _PH_ROWS * _C // (_GRP * _L)
_GPR = _C // (_GRP * _L)

_TC_BLK = 512


def _sc_partials_kernel(o_hbm, t_hbm, part_hbm,
                        ob, tb, pbuf, so0, st0, so1, st1):
    wid = lax.axis_index("s") * _NC + lax.axis_index("c")
    r0 = wid * _PW
    osems = (so0, so1)
    tsems = (st0, st1)

    def start(g, par):
        r = r0 + g * _PH_ROWS
        pltpu.async_copy(o_hbm.at[pl.ds(r, _PH_ROWS), :], ob.at[par],
                         osems[par])
        pltpu.async_copy(t_hbm.at[pl.ds(r, _PH_ROWS), :], tb.at[par],
                         tsems[par])

    def wait(par):
        pltpu.make_async_copy(o_hbm.at[pl.ds(0, _PH_ROWS), :], ob.at[par],
                              osems[par]).wait()
        pltpu.make_async_copy(t_hbm.at[pl.ds(0, _PH_ROWS), :], tb.at[par],
                              tsems[par]).wait()

    start(0, 0)
    start(1, 1)

    def group_body(par):
        def body(q, accs):
            acc_tp, acc_so, acc_st = accs
            row = q // _GPR
            col = (q % _GPR) * (_GRP * _L)
            for p in range(_GRP):
                o = ob[par, row, pl.ds(col + p * _L, _L)]
                t = tb[par, row, pl.ds(col + p * _L, _L)]
                m = o >= 0.5
                acc_so = acc_so + jnp.where(m, 1.0, 0.0)
                acc_st = acc_st + t
                acc_tp = acc_tp + jnp.where(m, t, 0.0)
            return acc_tp, acc_so, acc_st
        return body

    def phase_body(g, accs):
        par = lax.rem(g, 2)

        @pl.when(par == 0)
        def _():
            wait(0)

        @pl.when(par == 1)
        def _():
            wait(1)

        accs = lax.fori_loop(0, _NG, group_body(par), accs)

        nxt = g + 2

        @pl.when(jnp.logical_and(nxt < _NPH, par == 0))
        def _():
            start(nxt, 0)

        @pl.when(jnp.logical_and(nxt < _NPH, par == 1))
        def _():
            start(nxt, 1)

        return accs

    zeros = jnp.zeros((_L,), jnp.float32)
    acc_tp, acc_so, acc_st = lax.fori_loop(
        0, _NPH, phase_body, (zeros, zeros, zeros))

    pbuf[pl.ds(0, _L)] = acc_tp
    pbuf[pl.ds(_L, _L)] = acc_so
    pbuf[pl.ds(2 * _L, _L)] = acc_st
    pltpu.sync_copy(pbuf, part_hbm.at[wid])


def _tc_partials_kernel(o_ref, t_ref, acc_ref):
    i = pl.program_id(0)

    @pl.when(i == 0)
    def _():
        acc_ref[...] = jnp.zeros_like(acc_ref)

    o = o_ref[...]
    t = t_ref[...]
    m = o >= 0.5
    ob = jnp.where(m, 1.0, 0.0)
    tpv = jnp.where(m, t, 0.0)

    def red(v):
        return jnp.sum(v.reshape(_TC_BLK // 8, 8, _C // 128, 128),
                       axis=(0, 2))

    acc_ref[0] += red(tpv)
    acc_ref[1] += red(ob)
    acc_ref[2] += red(t)


_MG_NB = 4
_MG_RB = 512


def _tc_mega_kernel(nblk, blk0, o_hbm, t_hbm, acc_ref, obuf, tbuf,
                    osem, tsem):
    def start(b, slot):
        r = (blk0 + b) * _MG_RB
        pltpu.async_copy(o_hbm.at[pl.ds(r, _MG_RB), :], obuf.at[slot],
                         osem.at[slot])
        pltpu.async_copy(t_hbm.at[pl.ds(r, _MG_RB), :], tbuf.at[slot],
                         tsem.at[slot])

    def wait(slot):
        pltpu.make_async_copy(o_hbm.at[pl.ds(0, _MG_RB), :], obuf.at[slot],
                              osem.at[slot]).wait()
        pltpu.make_async_copy(t_hbm.at[pl.ds(0, _MG_RB), :], tbuf.at[slot],
                              tsem.at[slot]).wait()

    for s in range(min(_MG_NB, nblk)):
        start(s, s)

    def red(v):
        return jnp.sum(v.reshape(_MG_RB // 8, 8, _C // 128, 128),
                       axis=(0, 2))

    acc_tp = jnp.zeros((8, 128), jnp.float32)
    acc_so = jnp.zeros((8, 128), jnp.float32)
    acc_st = jnp.zeros((8, 128), jnp.float32)
    for b in range(nblk):
        slot = b % _MG_NB
        wait(slot)
        o = obuf[slot]
        t = tbuf[slot]
        m = o >= 0.5
        acc_so = acc_so + red(jnp.where(m, 1.0, 0.0))
        acc_st = acc_st + red(t)
        acc_tp = acc_tp + red(jnp.where(m, t, 0.0))
        if b + _MG_NB < nblk:
            start(b + _MG_NB, slot)

    acc_ref[0] = acc_tp
    acc_ref[1] = acc_so
    acc_ref[2] = acc_st


def _tc_mega(o2, t2, row_off, n_rows):
    return pl.pallas_call(
        functools.partial(_tc_mega_kernel, n_rows // _MG_RB,
                          row_off // _MG_RB),
        in_specs=[
            pl.BlockSpec(memory_space=pltpu.HBM),
            pl.BlockSpec(memory_space=pltpu.HBM),
        ],
        out_shape=jax.ShapeDtypeStruct((3, 8, 128), jnp.float32),
        scratch_shapes=[
            pltpu.VMEM((_MG_NB, _MG_RB, _C), jnp.float32),
            pltpu.VMEM((_MG_NB, _MG_RB, _C), jnp.float32),
            pltpu.SemaphoreType.DMA((_MG_NB,)),
            pltpu.SemaphoreType.DMA((_MG_NB,)),
        ],
    )(o2, t2)


def _tc_final_kernel(nblk, blk0, o_hbm, t_hbm, sc_ref, ca_ref, o_ref,
                     obuf, tbuf, osem, tsem):
    def start(b, slot):
        r = (blk0 + b) * _MG_RB
        pltpu.async_copy(o_hbm.at[pl.ds(r, _MG_RB), :], obuf.at[slot],
                         osem.at[slot])
        pltpu.async_copy(t_hbm.at[pl.ds(r, _MG_RB), :], tbuf.at[slot],
                         tsem.at[slot])

    def wait(slot):
        pltpu.make_async_copy(o_hbm.at[pl.ds(0, _MG_RB), :], obuf.at[slot],
                              osem.at[slot]).wait()
        pltpu.make_async_copy(t_hbm.at[pl.ds(0, _MG_RB), :], tbuf.at[slot],
                              tsem.at[slot]).wait()

    for s in range(min(_MG_NB, nblk)):
        start(s, s)

    def red(v):
        return jnp.sum(v.reshape(_MG_RB // 8, 8, _C // 128, 128),
                       axis=(0, 2))

    acc_tp = jnp.zeros((8, 128), jnp.float32)
    acc_so = jnp.zeros((8, 128), jnp.float32)
    acc_st = jnp.zeros((8, 128), jnp.float32)
    for b in range(nblk):
        slot = b % _MG_NB
        wait(slot)
        o = obuf[slot]
        t = tbuf[slot]
        m = o >= 0.5
        acc_so = acc_so + red(jnp.where(m, 1.0, 0.0))
        acc_st = acc_st + red(t)
        acc_tp = acc_tp + red(jnp.where(m, t, 0.0))
        if b + _MG_NB < nblk:
            start(b + _MG_NB, slot)

    s = sc_ref[...]
    c = ca_ref[...]
    tp = jnp.sum(s[:, 0:_L]) + jnp.sum(c[0]) + jnp.sum(acc_tp)
    sum_out = jnp.sum(s[:, _L:2 * _L]) + jnp.sum(c[1]) + jnp.sum(acc_so)
    sum_tgt = jnp.sum(s[:, 2 * _L:3 * _L]) + jnp.sum(c[2]) + jnp.sum(acc_st)
    fn = sum_tgt - tp
    fp = sum_out - tp
    recall = tp / (tp + fn)
    precision = tp / (tp + fp)
    f = ((1.0 + _BETA_SQUARED) * (precision * recall)
         / (_BETA_SQUARED * precision + recall))
    o_ref[...] = jnp.full((1, 1), f, jnp.float32)


def kernel(outputs, targets):
    o2 = outputs.reshape(_ROWS, _C)
    t2 = targets.reshape(_ROWS, _C)

    mesh = plsc.VectorSubcoreMesh(core_axis_name="c", subcore_axis_name="s",
                                  num_cores=_NC, num_subcores=_NS)
    sc_partials = pl.kernel(
        _sc_partials_kernel,
        out_type=jax.ShapeDtypeStruct((_NW, 3 * _L), jnp.float32),
        mesh=mesh,
        scratch_types=[
            pltpu.VMEM((2, _PH_ROWS, _C), jnp.float32),
            pltpu.VMEM((2, _PH_ROWS, _C), jnp.float32),
            pltpu.VMEM((3 * _L,), jnp.float32),
            pltpu.SemaphoreType.DMA,
            pltpu.SemaphoreType.DMA,
            pltpu.SemaphoreType.DMA,
            pltpu.SemaphoreType.DMA,
        ],
    )(o2, t2)

    tca_partials = _tc_mega(o2, t2, _SC_ROWS, _TCA_ROWS)

    f = pl.pallas_call(
        functools.partial(_tc_final_kernel, _TCB_ROWS // _MG_RB,
                          (_SC_ROWS + _TCA_ROWS) // _MG_RB),
        in_specs=[
            pl.BlockSpec(memory_space=pltpu.HBM),
            pl.BlockSpec(memory_space=pltpu.HBM),
            pl.BlockSpec((_NW, 3 * _L), lambda: (0, 0)),
            pl.BlockSpec((3, 8, 128), lambda: (0, 0, 0)),
        ],
        out_shape=jax.ShapeDtypeStruct((1, 1), jnp.float32),
        scratch_shapes=[
            pltpu.VMEM((_MG_NB, _MG_RB, _C), jnp.float32),
            pltpu.VMEM((_MG_NB, _MG_RB, _C), jnp.float32),
            pltpu.SemaphoreType.DMA((_MG_NB,)),
            pltpu.SemaphoreType.DMA((_MG_NB,)),
        ],
    )(o2, t2, sc_partials, tca_partials)
    return f.reshape(())


# speedup vs baseline: 1.1459x; 1.1459x over previous
import functools

import jax
import jax.numpy as jnp
from jax import lax
from jax.experimental import pallas as pl
from jax.experimental.pallas import tpu as pltpu
from jax.experimental.pallas import tpu_sc as plsc

_BETA_SQUARED = 1.0

_NC = 2
_NS = 16
_NW = _NC * _NS
_L = 16

_C = 512
_ROWS = 8192

_SC_ROWS = 4096
_TC_ROWS = _ROWS - _SC_ROWS

_PW = _SC_ROWS // _NW
_NPH = 4
_PH_ROWS = _PW // _NPH

_GRP = 8
_NG = _PH_ROWS * _C // (_GRP * _L)
_GPR = _C // (_GRP * _L)

_TC_BLK = 512


def _sc_partials_kernel(o_hbm, t_hbm, part_hbm,
                        ob, tb, pbuf, so0, st0, so1, st1):
    wid = lax.axis_index("s") * _NC + lax.axis_index("c")
    r0 = wid * _PW
    osems = (so0, so1)
    tsems = (st0, st1)

    def start(g, par):
        r = r0 + g * _PH_ROWS
        pltpu.async_copy(o_hbm.at[pl.ds(r, _PH_ROWS), :], ob.at[par],
                         osems[par])
        pltpu.async_copy(t_hbm.at[pl.ds(r, _PH_ROWS), :], tb.at[par],
                         tsems[par])

    def wait(par):
        pltpu.make_async_copy(o_hbm.at[pl.ds(0, _PH_ROWS), :], ob.at[par],
                              osems[par]).wait()
        pltpu.make_async_copy(t_hbm.at[pl.ds(0, _PH_ROWS), :], tb.at[par],
                              tsems[par]).wait()

    start(0, 0)
    start(1, 1)

    def group_body(par):
        def body(q, accs):
            acc_tp, acc_so, acc_st = accs
            row = q // _GPR
            col = (q % _GPR) * (_GRP * _L)
            for p in range(_GRP):
                o = ob[par, row, pl.ds(col + p * _L, _L)]
                t = tb[par, row, pl.ds(col + p * _L, _L)]
                m = o >= 0.5
                acc_so = acc_so + jnp.where(m, 1.0, 0.0)
                acc_st = acc_st + t
                acc_tp = acc_tp + jnp.where(m, t, 0.0)
            return acc_tp, acc_so, acc_st
        return body

    def phase_body(g, accs):
        par = lax.rem(g, 2)

        @pl.when(par == 0)
        def _():
            wait(0)

        @pl.when(par == 1)
        def _():
            wait(1)

        accs = lax.fori_loop(0, _NG, group_body(par), accs)

        nxt = g + 2

        @pl.when(jnp.logical_and(nxt < _NPH, par == 0))
        def _():
            start(nxt, 0)

        @pl.when(jnp.logical_and(nxt < _NPH, par == 1))
        def _():
            start(nxt, 1)

        return accs

    zeros = jnp.zeros((_L,), jnp.float32)
    acc_tp, acc_so, acc_st = lax.fori_loop(
        0, _NPH, phase_body, (zeros, zeros, zeros))

    pbuf[pl.ds(0, _L)] = acc_tp
    pbuf[pl.ds(_L, _L)] = acc_so
    pbuf[pl.ds(2 * _L, _L)] = acc_st
    pltpu.sync_copy(pbuf, part_hbm.at[wid])


def _tc_partials_kernel(o_ref, t_ref, acc_ref):
    i = pl.program_id(0)

    @pl.when(i == 0)
    def _():
        acc_ref[...] = jnp.zeros_like(acc_ref)

    o = o_ref[...]
    t = t_ref[...]
    m = o >= 0.5
    ob = jnp.where(m, 1.0, 0.0)
    tpv = jnp.where(m, t, 0.0)

    def red(v):
        return jnp.sum(v.reshape(_TC_BLK // 8, 8, _C // 128, 128),
                       axis=(0, 2))

    acc_ref[0] += red(tpv)
    acc_ref[1] += red(ob)
    acc_ref[2] += red(t)


_MG_NB = 4
_MG_RB = 512


def _tc_mega_kernel(nblk, blk0, o_hbm, t_hbm, acc_ref, obuf, tbuf,
                    osem, tsem):
    def start(b, slot):
        r = (blk0 + b) * _MG_RB
        pltpu.async_copy(o_hbm.at[pl.ds(r, _MG_RB), :], obuf.at[slot],
                         osem.at[slot])
        pltpu.async_copy(t_hbm.at[pl.ds(r, _MG_RB), :], tbuf.at[slot],
                         tsem.at[slot])

    def wait(slot):
        pltpu.make_async_copy(o_hbm.at[pl.ds(0, _MG_RB), :], obuf.at[slot],
                              osem.at[slot]).wait()
        pltpu.make_async_copy(t_hbm.at[pl.ds(0, _MG_RB), :], tbuf.at[slot],
                              tsem.at[slot]).wait()

    for s in range(min(_MG_NB, nblk)):
        start(s, s)

    def red(v):
        return jnp.sum(v.reshape(_MG_RB // 8, 8, _C // 128, 128),
                       axis=(0, 2))

    acc_tp = jnp.zeros((8, 128), jnp.float32)
    acc_so = jnp.zeros((8, 128), jnp.float32)
    acc_st = jnp.zeros((8, 128), jnp.float32)
    for b in range(nblk):
        slot = b % _MG_NB
        wait(slot)
        o = obuf[slot]
        t = tbuf[slot]
        m = o >= 0.5
        acc_so = acc_so + red(jnp.where(m, 1.0, 0.0))
        acc_st = acc_st + red(t)
        acc_tp = acc_tp + red(jnp.where(m, t, 0.0))
        if b + _MG_NB < nblk:
            start(b + _MG_NB, slot)

    acc_ref[0] = acc_tp
    acc_ref[1] = acc_so
    acc_ref[2] = acc_st


def _tc_mega(o2, t2, row_off, n_rows):
    return pl.pallas_call(
        functools.partial(_tc_mega_kernel, n_rows // _MG_RB,
                          row_off // _MG_RB),
        in_specs=[
            pl.BlockSpec(memory_space=pltpu.HBM),
            pl.BlockSpec(memory_space=pltpu.HBM),
        ],
        out_shape=jax.ShapeDtypeStruct((3, 8, 128), jnp.float32),
        scratch_shapes=[
            pltpu.VMEM((_MG_NB, _MG_RB, _C), jnp.float32),
            pltpu.VMEM((_MG_NB, _MG_RB, _C), jnp.float32),
            pltpu.SemaphoreType.DMA((_MG_NB,)),
            pltpu.SemaphoreType.DMA((_MG_NB,)),
        ],
    )(o2, t2)


def _finish_kernel(sc_ref, tc_ref, o_ref):
    s = sc_ref[...]
    c = tc_ref[...]
    tp = jnp.sum(s[:, 0:_L]) + jnp.sum(c[0])
    sum_out = jnp.sum(s[:, _L:2 * _L]) + jnp.sum(c[1])
    sum_tgt = jnp.sum(s[:, 2 * _L:3 * _L]) + jnp.sum(c[2])
    fn = sum_tgt - tp
    fp = sum_out - tp
    recall = tp / (tp + fn)
    precision = tp / (tp + fp)
    f = ((1.0 + _BETA_SQUARED) * (precision * recall)
         / (_BETA_SQUARED * precision + recall))
    o_ref[...] = jnp.full((1, 1), f, jnp.float32)


def kernel(outputs, targets):
    o2 = outputs.reshape(_ROWS, _C)
    t2 = targets.reshape(_ROWS, _C)

    mesh = plsc.VectorSubcoreMesh(core_axis_name="c", subcore_axis_name="s",
                                  num_cores=_NC, num_subcores=_NS)
    sc_partials = pl.kernel(
        _sc_partials_kernel,
        out_type=jax.ShapeDtypeStruct((_NW, 3 * _L), jnp.float32),
        mesh=mesh,
        scratch_types=[
            pltpu.VMEM((2, _PH_ROWS, _C), jnp.float32),
            pltpu.VMEM((2, _PH_ROWS, _C), jnp.float32),
            pltpu.VMEM((3 * _L,), jnp.float32),
            pltpu.SemaphoreType.DMA,
            pltpu.SemaphoreType.DMA,
            pltpu.SemaphoreType.DMA,
            pltpu.SemaphoreType.DMA,
        ],
    )(o2, t2)

    tc_partials = _tc_mega(o2, t2, _SC_ROWS, _TC_ROWS)

    f = pl.pallas_call(
        _finish_kernel,
        out_shape=jax.ShapeDtypeStruct((1, 1), jnp.float32),
    )(sc_partials, tc_partials)
    return f.reshape(())


# speedup vs baseline: 1.1463x; 1.0003x over previous
import functools

import jax
import jax.numpy as jnp
from jax import lax
from jax.experimental import pallas as pl
from jax.experimental.pallas import tpu as pltpu
from jax.experimental.pallas import tpu_sc as plsc

_BETA_SQUARED = 1.0

_NC = 2
_NS = 16
_NW = _NC * _NS
_L = 16

_C = 512
_ROWS = 8192

_SC_ROWS = 4096
_TC_ROWS = _ROWS - _SC_ROWS

_PW = _SC_ROWS // _NW
_NPH = 4
_PH_ROWS = _PW // _NPH

_GRP = 8
_NG = _PH_ROWS * _C // (_GRP * _L)
_GPR = _C // (_GRP * _L)

def _sc_partials_kernel(o_hbm, t_hbm, part_hbm,
                        ob, tb, pbuf, so0, st0, so1, st1):
    wid = lax.axis_index("s") * _NC + lax.axis_index("c")
    r0 = wid * _PW
    osems = (so0, so1)
    tsems = (st0, st1)

    def start(g, par):
        r = r0 + g * _PH_ROWS
        pltpu.async_copy(o_hbm.at[pl.ds(r, _PH_ROWS), :], ob.at[par],
                         osems[par])
        pltpu.async_copy(t_hbm.at[pl.ds(r, _PH_ROWS), :], tb.at[par],
                         tsems[par])

    def wait(par):
        pltpu.make_async_copy(o_hbm.at[pl.ds(0, _PH_ROWS), :], ob.at[par],
                              osems[par]).wait()
        pltpu.make_async_copy(t_hbm.at[pl.ds(0, _PH_ROWS), :], tb.at[par],
                              tsems[par]).wait()

    start(0, 0)
    start(1, 1)

    def group_body(par):
        def body(q, accs):
            acc_tp, acc_so, acc_st = accs
            row = q // _GPR
            col = (q % _GPR) * (_GRP * _L)
            for p in range(_GRP):
                o = ob[par, row, pl.ds(col + p * _L, _L)]
                t = tb[par, row, pl.ds(col + p * _L, _L)]
                m = o >= 0.5
                acc_so = acc_so + jnp.where(m, 1.0, 0.0)
                acc_st = acc_st + t
                acc_tp = acc_tp + jnp.where(m, t, 0.0)
            return acc_tp, acc_so, acc_st
        return body

    def phase_body(g, accs):
        par = lax.rem(g, 2)

        @pl.when(par == 0)
        def _():
            wait(0)

        @pl.when(par == 1)
        def _():
            wait(1)

        accs = lax.fori_loop(0, _NG, group_body(par), accs)

        nxt = g + 2

        @pl.when(jnp.logical_and(nxt < _NPH, par == 0))
        def _():
            start(nxt, 0)

        @pl.when(jnp.logical_and(nxt < _NPH, par == 1))
        def _():
            start(nxt, 1)

        return accs

    zeros = jnp.zeros((_L,), jnp.float32)
    acc_tp, acc_so, acc_st = lax.fori_loop(
        0, _NPH, phase_body, (zeros, zeros, zeros))

    pbuf[pl.ds(0, _L)] = acc_tp
    pbuf[pl.ds(_L, _L)] = acc_so
    pbuf[pl.ds(2 * _L, _L)] = acc_st
    pltpu.sync_copy(pbuf, part_hbm.at[wid])


_MG_NB = 4
_MG_RB = 512


def _tc_mega_kernel(nblk, blk0, o_hbm, t_hbm, acc_ref, obuf, tbuf,
                    osem, tsem):
    def start(b, slot):
        r = (blk0 + b) * _MG_RB
        pltpu.async_copy(o_hbm.at[pl.ds(r, _MG_RB), :], obuf.at[slot],
                         osem.at[slot])
        pltpu.async_copy(t_hbm.at[pl.ds(r, _MG_RB), :], tbuf.at[slot],
                         tsem.at[slot])

    def wait(slot):
        pltpu.make_async_copy(o_hbm.at[pl.ds(0, _MG_RB), :], obuf.at[slot],
                              osem.at[slot]).wait()
        pltpu.make_async_copy(t_hbm.at[pl.ds(0, _MG_RB), :], tbuf.at[slot],
                              tsem.at[slot]).wait()

    for s in range(min(_MG_NB, nblk)):
        start(s, s)

    def red(v):
        return jnp.sum(v.reshape(_MG_RB // 8, 8, _C // 128, 128),
                       axis=(0, 2))

    acc_tp = jnp.zeros((8, 128), jnp.float32)
    acc_so = jnp.zeros((8, 128), jnp.float32)
    acc_st = jnp.zeros((8, 128), jnp.float32)
    for b in range(nblk):
        slot = b % _MG_NB
        wait(slot)
        o = obuf[slot]
        t = tbuf[slot]
        m = o >= 0.5
        acc_so = acc_so + red(jnp.where(m, 1.0, 0.0))
        acc_st = acc_st + red(t)
        acc_tp = acc_tp + red(jnp.where(m, t, 0.0))
        if b + _MG_NB < nblk:
            start(b + _MG_NB, slot)

    acc_ref[0] = acc_tp
    acc_ref[1] = acc_so
    acc_ref[2] = acc_st


def _tc_mega(o2, t2, row_off, n_rows):
    return pl.pallas_call(
        functools.partial(_tc_mega_kernel, n_rows // _MG_RB,
                          row_off // _MG_RB),
        in_specs=[
            pl.BlockSpec(memory_space=pltpu.HBM),
            pl.BlockSpec(memory_space=pltpu.HBM),
        ],
        out_shape=jax.ShapeDtypeStruct((3, 8, 128), jnp.float32),
        scratch_shapes=[
            pltpu.VMEM((_MG_NB, _MG_RB, _C), jnp.float32),
            pltpu.VMEM((_MG_NB, _MG_RB, _C), jnp.float32),
            pltpu.SemaphoreType.DMA((_MG_NB,)),
            pltpu.SemaphoreType.DMA((_MG_NB,)),
        ],
    )(o2, t2)


def _finish_kernel(sc_ref, tc_ref, o_ref):
    s = sc_ref[...]
    c = tc_ref[...]
    tp = jnp.sum(s[:, 0:_L]) + jnp.sum(c[0])
    sum_out = jnp.sum(s[:, _L:2 * _L]) + jnp.sum(c[1])
    sum_tgt = jnp.sum(s[:, 2 * _L:3 * _L]) + jnp.sum(c[2])
    fn = sum_tgt - tp
    fp = sum_out - tp
    recall = tp / (tp + fn)
    precision = tp / (tp + fp)
    f = ((1.0 + _BETA_SQUARED) * (precision * recall)
         / (_BETA_SQUARED * precision + recall))
    o_ref[...] = jnp.full((1, 1), f, jnp.float32)


def kernel(outputs, targets):
    o2 = outputs.reshape(_ROWS, _C)
    t2 = targets.reshape(_ROWS, _C)

    mesh = plsc.VectorSubcoreMesh(core_axis_name="c", subcore_axis_name="s",
                                  num_cores=_NC, num_subcores=_NS)
    sc_partials = pl.kernel(
        _sc_partials_kernel,
        out_type=jax.ShapeDtypeStruct((_NW, 3 * _L), jnp.float32),
        mesh=mesh,
        scratch_types=[
            pltpu.VMEM((2, _PH_ROWS, _C), jnp.float32),
            pltpu.VMEM((2, _PH_ROWS, _C), jnp.float32),
            pltpu.VMEM((3 * _L,), jnp.float32),
            pltpu.SemaphoreType.DMA,
            pltpu.SemaphoreType.DMA,
            pltpu.SemaphoreType.DMA,
            pltpu.SemaphoreType.DMA,
        ],
    )(o2, t2)

    tc_partials = _tc_mega(o2, t2, _SC_ROWS, _TC_ROWS)

    f = pl.pallas_call(
        _finish_kernel,
        out_shape=jax.ShapeDtypeStruct((1, 1), jnp.float32),
    )(sc_partials, tc_partials)
    return f.reshape(())
